# Initial kernel scaffold; baseline (speedup 1.0000x reference)
import jax
import jax.numpy as jnp
from jax.experimental import pallas as pl


def kernel(adj, nodes, node_type_mask, node_pos_matrix, emb, pos_emb, type_emb, w_pos_type, gcn_w, gcn_b):
    raise NotImplementedError("write your pallas kernel here")


# speedup vs baseline: 3.0856x; 3.0856x over previous
import functools

import jax
import jax.numpy as jnp
from jax import lax
from jax.experimental import pallas as pl
from jax.experimental.pallas import tpu as pltpu
from jax.experimental.pallas import tpu_sc as plsc

DIM = 128
N = 200
L = 50
N_LAYER = 2
BB = 8


def _sc_gather(emb, idx_flat):
    info = plsc.get_sparse_core_info()
    nw = info.num_cores * info.num_subcores
    total = idx_flat.shape[0]
    per_w = total // nw
    chunk = 128
    n_chunks = per_w // chunk

    mesh = plsc.VectorSubcoreMesh(core_axis_name="c", subcore_axis_name="s")

    @functools.partial(
        pl.kernel,
        mesh=mesh,
        out_type=jax.ShapeDtypeStruct((total, DIM), jnp.float32),
        scratch_types=[
            pltpu.VMEM((2, chunk), jnp.int32),
            pltpu.VMEM((2, chunk, DIM), jnp.float32),
            pltpu.SemaphoreType.DMA,
        ],
    )
    def k(idx_hbm, table_hbm, out_hbm, idx_v, rows_v, gsem):
        wid = lax.axis_index("s") * info.num_cores + lax.axis_index("c")
        w_base = wid * per_w

        def start(i, slot):
            base = w_base + i * chunk
            pltpu.sync_copy(idx_hbm.at[pl.ds(base, chunk)], idx_v.at[slot])
            pltpu.async_copy(table_hbm.at[idx_v.at[slot]], rows_v.at[slot], gsem)

        def finish(i, slot):
            base = w_base + i * chunk
            pltpu.make_async_copy(
                table_hbm.at[idx_v.at[slot]], rows_v.at[slot], gsem
            ).wait()
            pltpu.sync_copy(rows_v.at[slot], out_hbm.at[pl.ds(base, chunk)])

        start(0, 0)

        def body(g, c):
            i = g * 2
            @pl.when(i + 1 < n_chunks)
            def _():
                start(i + 1, 1)

            finish(i, 0)

            @pl.when(i + 2 < n_chunks)
            def _():
                start(i + 2, 0)

            @pl.when(i + 1 < n_chunks)
            def _():
                finish(i + 1, 1)

            return c

        lax.fori_loop(0, (n_chunks + 1) // 2, body, 0)

    return k(idx_flat, emb)


def _tc_body(hg_ref, adj_ref, mask_ref, posm_ref, pe_ref, te_ref, w_ref,
             gw_ref, gb_ref, out_ref):
    mask = mask_ref[...]
    clamp = jnp.minimum(mask, 1).astype(jnp.float32)
    onehot = (mask == lax.broadcasted_iota(jnp.int32, (1, 4), 1)).astype(
        jnp.float32
    )
    type_e = jnp.dot(onehot, te_ref[...], preferred_element_type=jnp.float32)

    pos = posm_ref[...].reshape(BB * N, L)
    pos_num = jnp.sum(pos, axis=1, keepdims=True)
    pos_e = jnp.dot(pos, pe_ref[...], preferred_element_type=jnp.float32)
    pos_e = pos_e / (pos_num + 1e-9) * clamp

    hg = hg_ref[...].reshape(BB * N, DIM)
    h = jnp.dot(
        jnp.concatenate([hg, type_e, pos_e], axis=1),
        w_ref[...],
        preferred_element_type=jnp.float32,
    )

    gw = gw_ref[...]
    gb = gb_ref[...]
    for _ in range(N_LAYER):
        pieces = []
        for b in range(BB):
            hb = h[b * N:(b + 1) * N]
            t = jnp.dot(adj_ref[b], hb, preferred_element_type=jnp.float32)
            t = jnp.dot(t, gw, preferred_element_type=jnp.float32)
            pieces.append(t)
        h = jnp.maximum(jnp.concatenate(pieces, axis=0) + gb, 0.0) * clamp
    out_ref[...] = h.reshape(BB, N, DIM)


def _tc_pipeline(hg, adj, mask, posm, pe, te, w, gw, gb):
    b_total = adj.shape[0]
    grid = (b_total // BB,)
    return pl.pallas_call(
        _tc_body,
        grid=grid,
        in_specs=[
            pl.BlockSpec((BB, N, DIM), lambda i: (i, 0, 0)),
            pl.BlockSpec((BB, N, N), lambda i: (i, 0, 0)),
            pl.BlockSpec((BB * N, 1), lambda i: (i, 0)),
            pl.BlockSpec((BB, N, L), lambda i: (i, 0, 0)),
            pl.BlockSpec((L, DIM), lambda i: (0, 0)),
            pl.BlockSpec((4, DIM), lambda i: (0, 0)),
            pl.BlockSpec((3 * DIM, DIM), lambda i: (0, 0)),
            pl.BlockSpec((DIM, DIM), lambda i: (0, 0)),
            pl.BlockSpec((1, DIM), lambda i: (0, 0)),
        ],
        out_specs=pl.BlockSpec((BB, N, DIM), lambda i: (i, 0, 0)),
        out_shape=jax.ShapeDtypeStruct((b_total, N, DIM), jnp.float32),
    )(hg, adj, mask, posm, pe, te, w, gw, gb)


def kernel(adj, nodes, node_type_mask, node_pos_matrix, emb, pos_emb,
           type_emb, w_pos_type, gcn_w, gcn_b):
    b_total, n = nodes.shape
    idx_flat = nodes.reshape(-1).astype(jnp.int32)
    hg = _sc_gather(emb, idx_flat).reshape(b_total, n, DIM)
    return _tc_pipeline(
        hg,
        adj,
        node_type_mask.reshape(b_total * n, 1),
        node_pos_matrix,
        pos_emb[:L],
        type_emb,
        w_pos_type,
        gcn_w,
        gcn_b.reshape(1, DIM),
    )


# speedup vs baseline: 3.3214x; 1.0764x over previous
import functools

import jax
import jax.numpy as jnp
from jax import lax
from jax.experimental import pallas as pl
from jax.experimental.pallas import tpu as pltpu
from jax.experimental.pallas import tpu_sc as plsc

DIM = 128
N = 200
L = 50
N_LAYER = 2
BB = 16


def _sc_gather(emb, idx_flat):
    info = plsc.get_sparse_core_info()
    nw = info.num_cores * info.num_subcores
    total = idx_flat.shape[0]
    per_w = total // nw
    chunk = 128
    n_chunks = per_w // chunk

    mesh = plsc.VectorSubcoreMesh(core_axis_name="c", subcore_axis_name="s")

    @functools.partial(
        pl.kernel,
        mesh=mesh,
        out_type=jax.ShapeDtypeStruct((total, DIM), jnp.float32),
        scratch_types=[
            pltpu.VMEM((2, chunk), jnp.int32),
            pltpu.VMEM((2, chunk, DIM), jnp.float32),
            pltpu.SemaphoreType.DMA,
        ],
    )
    def k(idx_hbm, table_hbm, out_hbm, idx_v, rows_v, gsem):
        wid = lax.axis_index("s") * info.num_cores + lax.axis_index("c")
        w_base = wid * per_w

        def start(i, slot):
            base = w_base + i * chunk
            pltpu.sync_copy(idx_hbm.at[pl.ds(base, chunk)], idx_v.at[slot])
            pltpu.async_copy(table_hbm.at[idx_v.at[slot]], rows_v.at[slot], gsem)

        def finish(i, slot):
            base = w_base + i * chunk
            pltpu.make_async_copy(
                table_hbm.at[idx_v.at[slot]], rows_v.at[slot], gsem
            ).wait()
            pltpu.sync_copy(rows_v.at[slot], out_hbm.at[pl.ds(base, chunk)])

        start(0, 0)

        def body(g, c):
            i = g * 2
            @pl.when(i + 1 < n_chunks)
            def _():
                start(i + 1, 1)

            finish(i, 0)

            @pl.when(i + 2 < n_chunks)
            def _():
                start(i + 2, 0)

            @pl.when(i + 1 < n_chunks)
            def _():
                finish(i + 1, 1)

            return c

        lax.fori_loop(0, (n_chunks + 1) // 2, body, 0)

    return k(idx_flat, emb)


def _tc_body(hg_ref, adj_ref, mask_ref, posm_ref, pe_ref, te_ref, w_ref,
             gw_ref, gb_ref, out_ref):
    mask = mask_ref[...]
    clamp = jnp.minimum(mask, 1).astype(jnp.float32)
    onehot = (mask == lax.broadcasted_iota(jnp.int32, (1, 4), 1)).astype(
        jnp.float32
    )
    type_e = jnp.dot(onehot, te_ref[...], preferred_element_type=jnp.float32)

    pos = posm_ref[...].reshape(BB * N, L)
    pos_num = jnp.sum(pos, axis=1, keepdims=True)
    pos_e = jnp.dot(pos, pe_ref[...], preferred_element_type=jnp.float32)
    pos_e = pos_e / (pos_num + 1e-9) * clamp

    hg = hg_ref[...].reshape(BB * N, DIM)
    h = jnp.dot(
        jnp.concatenate([hg, type_e, pos_e], axis=1),
        w_ref[...],
        preferred_element_type=jnp.float32,
    )

    gw = gw_ref[...]
    gb = gb_ref[...]
    for _ in range(N_LAYER):
        pieces = []
        for b in range(BB):
            hb = h[b * N:(b + 1) * N]
            t = jnp.dot(adj_ref[b], hb, preferred_element_type=jnp.float32)
            t = jnp.dot(t, gw, preferred_element_type=jnp.float32)
            pieces.append(t)
        h = jnp.maximum(jnp.concatenate(pieces, axis=0) + gb, 0.0) * clamp
    out_ref[...] = h.reshape(BB, N, DIM)


def _tc_pipeline(hg, adj, mask, posm, pe, te, w, gw, gb):
    b_total = adj.shape[0]
    grid = (b_total // BB,)
    return pl.pallas_call(
        _tc_body,
        grid=grid,
        in_specs=[
            pl.BlockSpec((BB, N, DIM), lambda i: (i, 0, 0)),
            pl.BlockSpec((BB, N, N), lambda i: (i, 0, 0)),
            pl.BlockSpec((BB * N, 1), lambda i: (i, 0)),
            pl.BlockSpec((BB, N, L), lambda i: (i, 0, 0)),
            pl.BlockSpec((L, DIM), lambda i: (0, 0)),
            pl.BlockSpec((4, DIM), lambda i: (0, 0)),
            pl.BlockSpec((3 * DIM, DIM), lambda i: (0, 0)),
            pl.BlockSpec((DIM, DIM), lambda i: (0, 0)),
            pl.BlockSpec((1, DIM), lambda i: (0, 0)),
        ],
        out_specs=pl.BlockSpec((BB, N, DIM), lambda i: (i, 0, 0)),
        out_shape=jax.ShapeDtypeStruct((b_total, N, DIM), jnp.float32),
    )(hg, adj, mask, posm, pe, te, w, gw, gb)


def kernel(adj, nodes, node_type_mask, node_pos_matrix, emb, pos_emb,
           type_emb, w_pos_type, gcn_w, gcn_b):
    b_total, n = nodes.shape
    idx_flat = nodes.reshape(-1).astype(jnp.int32)
    hg = _sc_gather(emb, idx_flat).reshape(b_total, n, DIM)
    return _tc_pipeline(
        hg,
        adj,
        node_type_mask.reshape(b_total * n, 1),
        node_pos_matrix,
        pos_emb[:L],
        type_emb,
        w_pos_type,
        gcn_w,
        gcn_b.reshape(1, DIM),
    )
